# Initial kernel scaffold; baseline (speedup 1.0000x reference)
import jax
import jax.numpy as jnp
from jax.experimental import pallas as pl


def kernel(encodings, subnetwork, W, b):
    raise NotImplementedError("write your pallas kernel here")


# speedup vs baseline: 15.3747x; 15.3747x over previous
import functools

import jax
import jax.numpy as jnp
from jax import lax
from jax.experimental import pallas as pl
from jax.experimental.pallas import tpu as pltpu
from jax.experimental.pallas import tpu_sc as plsc

N = 10000
D = 128
NC = 2
NS = 16
NW = NC * NS
CHUNK = 128
CPW = 80
G = 16
EPAD = NW * CPW * CHUNK
NPAD = 10112
TRASH = N
RPT = NPAD // NS
OPT = 624
OTAIL = N - NS * OPT

_MESH = plsc.VectorSubcoreMesh(core_axis_name="c", subcore_axis_name="s")


@functools.partial(
    pl.kernel,
    out_type=jax.ShapeDtypeStruct((NPAD,), jnp.float32),
    mesh=_MESH,
    scratch_types=[
        pltpu.VMEM((2 * CPW, CHUNK), jnp.int32),
        pltpu.VMEM((CHUNK,), jnp.float32),
        pltpu.VMEM((640,), jnp.float32),
        pltpu.VMEM_SHARED((NPAD,), jnp.float32),
        pltpu.SemaphoreType.DMA,
    ],
)
def _deg_kernel(col_hbm, deg_out, colv, ones, stage, dacc, sem):
    c = lax.axis_index("c")
    s = lax.axis_index("s")

    @pl.when(c == 0)
    def _():
        for i in range(640 // 16):
            stage[pl.ds(i * 16, 16)] = jnp.zeros((16,), jnp.float32)
        for i in range(CHUNK // 16):
            ones[pl.ds(i * 16, 16)] = jnp.ones((16,), jnp.float32)
        pltpu.sync_copy(stage.at[pl.ds(0, RPT)], dacc.at[pl.ds(s * RPT, RPT)])
        pltpu.sync_copy(col_hbm.at[2 * s], colv.at[pl.ds(0, CPW)])
        pltpu.sync_copy(col_hbm.at[2 * s + 1], colv.at[pl.ds(CPW, CPW)])
        plsc.subcore_barrier()

        @pl.loop(0, 2 * CPW)
        def _(j):
            pltpu.async_copy(ones, dacc.at[colv.at[j]], sem, add=True)

        @pl.loop(0, 2 * CPW)
        def _(j):
            pltpu.make_async_copy(ones, dacc.at[colv.at[0]], sem).wait()

        plsc.subcore_barrier()
        pltpu.sync_copy(dacc.at[pl.ds(s * RPT, RPT)], stage.at[pl.ds(0, RPT)])
        pltpu.sync_copy(stage.at[pl.ds(0, RPT)],
                        deg_out.at[pl.ds(s * RPT, RPT)])


@functools.partial(
    pl.kernel,
    out_type=jax.ShapeDtypeStruct((NC, N, D), jnp.float32),
    mesh=_MESH,
    scratch_types=[
        pltpu.VMEM((G, CHUNK), jnp.int32),
        pltpu.VMEM((G, CHUNK), jnp.int32),
        pltpu.VMEM((CHUNK, D), jnp.float32),
        pltpu.VMEM((CHUNK, D), jnp.float32),
        pltpu.VMEM_SHARED((NPAD, D), jnp.float32),
        pltpu.SemaphoreType.DMA,
        pltpu.SemaphoreType.DMA,
    ],
)
def _msg_kernel(hs_hbm, row_hbm, col_hbm, m_out, rowv, colv, g0, g1, macc,
                sem0, sem1):
    c = lax.axis_index("c")
    s = lax.axis_index("s")
    wid = c * NS + s

    @pl.loop(0, CHUNK)
    def _(i):
        for l in range(D // 16):
            g0[i, pl.ds(l * 16, 16)] = jnp.zeros((16,), jnp.float32)

    for t in range(4):
        pltpu.sync_copy(g0, macc.at[pl.ds(s * RPT + t * CHUNK, CHUNK)])
    pltpu.sync_copy(g0.at[pl.ds(0, RPT - 4 * CHUNK)],
                    macc.at[pl.ds(s * RPT + 4 * CHUNK, RPT - 4 * CHUNK)])
    plsc.subcore_barrier()

    for blk in range(CPW // G):
        pltpu.sync_copy(row_hbm.at[wid, pl.ds(blk * G, G)], rowv)
        pltpu.sync_copy(col_hbm.at[wid, pl.ds(blk * G, G)], colv)

        @pl.loop(0, G // 2)
        def _(j):
            a0 = pltpu.async_copy(hs_hbm.at[rowv.at[2 * j]], g0, sem0)
            a1 = pltpu.async_copy(hs_hbm.at[rowv.at[2 * j + 1]], g1, sem1)
            a0.wait()
            pltpu.sync_copy(g0, macc.at[colv.at[2 * j]], add=True)
            a1.wait()
            pltpu.sync_copy(g1, macc.at[colv.at[2 * j + 1]], add=True)

    plsc.subcore_barrier()
    pltpu.sync_copy(macc.at[pl.ds(s * OPT, OPT)],
                    m_out.at[c, pl.ds(s * OPT, OPT)])

    @pl.when(s == 0)
    def _():
        pltpu.sync_copy(macc.at[pl.ds(NS * OPT, OTAIL)],
                        m_out.at[c, pl.ds(NS * OPT, OTAIL)])


def _mm_body(x_ref, w_ref, deg_ref, hs_ref):
    dinv = lax.rsqrt(deg_ref[...] + 1.0)
    h = jnp.dot(x_ref[...], w_ref[...], preferred_element_type=jnp.float32)
    hs_ref[...] = dinv * h


def _out_body(mp_ref, hs_ref, deg_ref, b_ref, out_ref):
    dinv = lax.rsqrt(deg_ref[...] + 1.0)
    m = mp_ref[0] + mp_ref[1] + hs_ref[...]
    out_ref[...] = dinv * m + b_ref[...][None, :]


def kernel(encodings, subnetwork, W, b):
    E = subnetwork.shape[1]
    row = subnetwork[0]
    col = subnetwork[1]
    rowp = jnp.concatenate(
        [row, jnp.zeros((EPAD - E,), jnp.int32)]).reshape(NW, CPW, CHUNK)
    colp = jnp.concatenate(
        [col, jnp.full((EPAD - E,), TRASH, jnp.int32)]).reshape(NW, CPW, CHUNK)

    deg = _deg_kernel(colp)
    deg_col = deg[:N].reshape(N, 1)

    hs = pl.pallas_call(
        _mm_body,
        out_shape=jax.ShapeDtypeStruct((N, D), jnp.float32),
    )(encodings, W, deg_col)

    mp = _msg_kernel(hs, rowp, colp)

    out = pl.pallas_call(
        _out_body,
        out_shape=jax.ShapeDtypeStruct((N, D), jnp.float32),
    )(mp, hs, deg_col, b)
    return out
